# baseline (speedup 1.0000x reference)
import functools

import jax
import jax.numpy as jnp
from jax import lax
from jax.experimental import pallas as pl
from jax.experimental.pallas import tpu as pltpu
from jax.experimental.pallas import tpu_sc as plsc

D_MODEL = 64
NUM_CORES = 2
NUM_SUBCORES = 16
NUM_WORKERS = NUM_CORES * NUM_SUBCORES
CHUNK = 512
NBUF = 2


def _emb_call(B, b_per_w, n_chunks):
    mesh = plsc.VectorSubcoreMesh(core_axis_name="c", subcore_axis_name="s")

    @functools.partial(
        pl.kernel,
        mesh=mesh,
        compiler_params=pltpu.CompilerParams(use_tc_tiling_on_sc=False),
        out_type=jax.ShapeDtypeStruct((B, D_MODEL), jnp.float32),
        scratch_types=[
            pltpu.VMEM((n_chunks, CHUNK), jnp.int32),
            pltpu.VMEM((NBUF, CHUNK, D_MODEL), jnp.float32),
            pltpu.SemaphoreType.DMA((NBUF,)),
        ],
    )
    def emb(table_hbm, idx_hbm, out_hbm, idx_v, rows_v, gsem):
        wid = lax.axis_index("s") * NUM_CORES + lax.axis_index("c")
        base = wid * b_per_w
        pltpu.sync_copy(idx_hbm.at[wid], idx_v)

        def gather(g, b):
            return pltpu.make_async_copy(
                table_hbm.at[idx_v.at[g]], rows_v.at[b], gsem.at[b])

        for b in range(NBUF):
            gather(b, b).start()

        def body(i, carry):
            for b in range(NBUF):
                g = i * NBUF + b
                gather(g, b).wait()
                pltpu.sync_copy(
                    rows_v.at[b], out_hbm.at[pl.ds(base + g * CHUNK, CHUNK)])
                gather(g + NBUF, b).start()
            return carry

        n_outer = n_chunks // NBUF
        lax.fori_loop(0, n_outer - 1, body, 0)
        for b in range(NBUF):
            g = (n_outer - 1) * NBUF + b
            gather(g, b).wait()
            pltpu.sync_copy(
                rows_v.at[b], out_hbm.at[pl.ds(base + g * CHUNK, CHUNK)])

    return emb


def kernel(x, W):
    n_rows, seq = x.shape
    B = n_rows * seq
    b_per_w = B // NUM_WORKERS
    n_chunks = b_per_w // CHUNK
    idx = x.reshape(NUM_WORKERS, n_chunks, CHUNK).astype(jnp.int32)
    out = _emb_call(B, b_per_w, n_chunks)(W, idx)
    return out.reshape(n_rows, seq, D_MODEL)


# speedup vs baseline: 1.0005x; 1.0005x over previous
import functools

import jax
import jax.numpy as jnp
from jax import lax
from jax.experimental import pallas as pl
from jax.experimental.pallas import tpu as pltpu
from jax.experimental.pallas import tpu_sc as plsc

D_MODEL = 64
NUM_CORES = 2
NUM_SUBCORES = 16
NUM_WORKERS = NUM_CORES * NUM_SUBCORES
NBUF = 4


def _emb_call(n_rows, seq):
    rows_per_w = n_rows // NUM_WORKERS
    mesh = plsc.VectorSubcoreMesh(core_axis_name="c", subcore_axis_name="s")

    @functools.partial(
        pl.kernel,
        mesh=mesh,
        compiler_params=pltpu.CompilerParams(use_tc_tiling_on_sc=False),
        out_type=jax.ShapeDtypeStruct((n_rows, seq, D_MODEL), jnp.float32),
        scratch_types=[
            pltpu.VMEM((rows_per_w, seq), jnp.int32),
            pltpu.VMEM((NBUF, seq, D_MODEL), jnp.float32),
            pltpu.SemaphoreType.DMA((NBUF,)),
        ],
    )
    def emb(table_hbm, idx_hbm, out_hbm, idx_v, rows_v, gsem):
        wid = lax.axis_index("s") * NUM_CORES + lax.axis_index("c")
        base = wid * rows_per_w
        pltpu.sync_copy(idx_hbm.at[pl.ds(base, rows_per_w)], idx_v)

        def gather(g, b):
            return pltpu.make_async_copy(
                table_hbm.at[idx_v.at[g]], rows_v.at[b], gsem.at[b])

        for b in range(NBUF):
            gather(b, b).start()

        def body(i, carry):
            for b in range(NBUF):
                g = i * NBUF + b
                gather(g, b).wait()
                pltpu.sync_copy(rows_v.at[b], out_hbm.at[base + g])
                gather(g + NBUF, b).start()
            return carry

        n_outer = rows_per_w // NBUF
        lax.fori_loop(0, n_outer - 1, body, 0)
        for b in range(NBUF):
            g = (n_outer - 1) * NBUF + b
            gather(g, b).wait()
            pltpu.sync_copy(rows_v.at[b], out_hbm.at[base + g])

    return emb


def kernel(x, W):
    n_rows, seq = x.shape
    return _emb_call(n_rows, seq)(W, x.astype(jnp.int32))


# speedup vs baseline: 1.0493x; 1.0488x over previous
import functools

import jax
import jax.numpy as jnp
from jax import lax
from jax.experimental import pallas as pl
from jax.experimental.pallas import tpu as pltpu
from jax.experimental.pallas import tpu_sc as plsc

D_MODEL = 64
NUM_CORES = 2
NUM_SUBCORES = 16
NUM_WORKERS = NUM_CORES * NUM_SUBCORES
CHUNK = 128
NBUF = 2
UNROLL = 16


def _emb_call(B):
    n_chunks = B // (NUM_WORKERS * CHUNK)
    mesh = plsc.VectorSubcoreMesh(core_axis_name="c", subcore_axis_name="s")

    @functools.partial(
        pl.kernel,
        mesh=mesh,
        compiler_params=pltpu.CompilerParams(use_tc_tiling_on_sc=True),
        out_type=jax.ShapeDtypeStruct((B, D_MODEL), jnp.float32),
        scratch_types=[
            pltpu.VMEM((n_chunks, CHUNK), jnp.int32),
            pltpu.VMEM((n_chunks, CHUNK), jnp.int32),
            pltpu.VMEM((NBUF, CHUNK, 2 * D_MODEL), jnp.float32),
            pltpu.VMEM((NBUF, CHUNK, D_MODEL), jnp.float32),
            pltpu.SemaphoreType.DMA((NBUF,)),
            pltpu.SemaphoreType.DMA((NBUF,)),
        ],
    )
    def emb(table_hbm, sidx_hbm, hoff_hbm, out_hbm,
            sidx_v, hoff_v, g_v, o_v, gsem, wsem):
        wid = lax.axis_index("s") * NUM_CORES + lax.axis_index("c")
        base = wid * n_chunks
        pltpu.sync_copy(sidx_hbm.at[wid], sidx_v)
        pltpu.sync_copy(hoff_hbm.at[wid], hoff_v)

        def gather(g, b):
            return pltpu.make_async_copy(
                table_hbm.at[sidx_v.at[g]], g_v.at[b], gsem.at[b])

        def out_write(g, b):
            return pltpu.make_async_copy(
                o_v.at[b], out_hbm.at[pl.ds((base + g) * CHUNK, CHUNK)],
                wsem.at[b])

        def compact(g, b):
            def blk_body(k, carry):
                kk = k * UNROLL
                hv = hoff_v[g, pl.ds(kk, UNROLL)]
                for r in range(UNROLL):
                    row = kk + r
                    h = hv[r]
                    for q in range(D_MODEL // 16):
                        o_v[b, row, pl.ds(q * 16, 16)] = (
                            g_v[b, row, pl.ds(h + q * 16, 16)])
                return carry

            lax.fori_loop(0, CHUNK // UNROLL, blk_body, 0)

        def step(g, b, wait_prev):
            gather(g, b).wait()
            if wait_prev:
                out_write(g - NBUF, b).wait()
            compact(g, b)
            out_write(g, b).start()

        for b in range(NBUF):
            gather(b, b).start()

        def body(i, carry):
            for b in range(NBUF):
                g = i * NBUF + b
                step(g, b, True)
                gather(g + NBUF, b).start()
            return carry

        for b in range(NBUF):
            step(b, b, False)
            gather(b + NBUF, b).start()
        n_outer = n_chunks // NBUF
        lax.fori_loop(1, n_outer - 1, body, 0)
        for b in range(NBUF):
            step((n_outer - 1) * NBUF + b, b, True)
        for b in range(NBUF):
            out_write((n_outer - 1) * NBUF + b, b).wait()

    return emb


def kernel(x, W):
    n_rows, seq = x.shape
    B = n_rows * seq
    xi = x.astype(jnp.int32).reshape(NUM_WORKERS, -1, CHUNK)
    sidx = xi >> 1
    hoff = (xi & 1) << 6
    table = W.reshape(-1, 2 * D_MODEL)
    out = _emb_call(B)(table, sidx, hoff)
    return out.reshape(n_rows, seq, D_MODEL)
